# baseline (speedup 1.0000x reference)
import functools
import jax
import jax.numpy as jnp
from jax import lax
from jax.experimental import pallas as pl
from jax.experimental.pallas import tpu as pltpu
from jax.experimental.pallas import tpu_sc as plsc

_TB = 2048
_NE = 64
_NC = 2
_NS = 16
_NW = _NC * _NS
_TPW = 512
_L = 16
_UNROLL = 8


def _logits_body(x_ref, w_ref, b_ref, out_ref):
    x = x_ref[...]
    w = w_ref[...]
    lt = jax.lax.dot_general(
        w, x, (((1,), (1,)), ((), ())), preferred_element_type=jnp.float32
    )
    lt = lt + b_ref[...]
    for w_local in range(_TB // _TPW):
        out_ref[w_local] = lt[:, w_local * _TPW:(w_local + 1) * _TPW]


def _tc_logits(hidden_states, gate_w, gate_b):
    T, H = hidden_states.shape
    wpb = _TB // _TPW
    return pl.pallas_call(
        _logits_body,
        grid=(T // _TB,),
        in_specs=[
            pl.BlockSpec((_TB, H), lambda i: (i, 0)),
            pl.BlockSpec((_NE, H), lambda i: (0, 0)),
            pl.BlockSpec((_NE, 1), lambda i: (0, 0)),
        ],
        out_specs=pl.BlockSpec((wpb, _NE, _TPW), lambda i: (i, 0, 0)),
        out_shape=jax.ShapeDtypeStruct((T // _TPW, _NE, _TPW), jnp.float32),
    )(hidden_states, gate_w, gate_b.reshape(_NE, 1))


def _make_sc_router(T):
    mesh = plsc.VectorSubcoreMesh(core_axis_name="c", subcore_axis_name="s")

    @functools.partial(
        pl.kernel,
        mesh=mesh,
        out_type=[
            jax.ShapeDtypeStruct((T,), jnp.float32),
            jax.ShapeDtypeStruct((T,), jnp.float32),
            jax.ShapeDtypeStruct((T,), jnp.int32),
            jax.ShapeDtypeStruct((T,), jnp.int32),
        ],
        scratch_types=[
            pltpu.VMEM((_NE, _TPW), jnp.float32),
            pltpu.VMEM((_TPW,), jnp.float32),
            pltpu.VMEM((_TPW,), jnp.float32),
            pltpu.VMEM((_TPW,), jnp.int32),
            pltpu.VMEM((_TPW,), jnp.int32),
        ],
    )
    def sc_router(logits_hbm, w1_hbm, w2_hbm, i1_hbm, i2_hbm,
                  chunk, w1v, w2v, i1v, i2v):
        wid = lax.axis_index("s") * _NC + lax.axis_index("c")
        base = wid * _TPW
        pltpu.sync_copy(logits_hbm.at[wid], chunk)

        def group(g, _):
            ga = g * 2 * _L
            gb = ga + _L
            neg = jnp.full((_L,), -jnp.inf, jnp.float32)
            zz = jnp.zeros((_L,), jnp.int32)
            ones = jnp.ones((_L,), jnp.int32)

            def estep(k, c):
                m1a, m2a, j1a, j2a, m1b, m2b, j1b, j2b, ev = c
                for d in range(_UNROLL):
                    e = k * _UNROLL + d
                    va = chunk[e, pl.ds(ga, _L)]
                    vb = chunk[e, pl.ds(gb, _L)]
                    gt1a = va > m1a
                    gt2a = va > m2a
                    gt1b = vb > m1b
                    gt2b = vb > m2b
                    m2a_n = jnp.maximum(m2a, jnp.minimum(m1a, va))
                    m2b_n = jnp.maximum(m2b, jnp.minimum(m1b, vb))
                    j2a = jnp.where(gt1a, j1a, jnp.where(gt2a, ev, j2a))
                    j2b = jnp.where(gt1b, j1b, jnp.where(gt2b, ev, j2b))
                    m1a, m2a = jnp.maximum(m1a, va), m2a_n
                    m1b, m2b = jnp.maximum(m1b, vb), m2b_n
                    j1a = jnp.where(gt1a, ev, j1a)
                    j1b = jnp.where(gt1b, ev, j1b)
                    ev = ev + ones
                return (m1a, m2a, j1a, j2a, m1b, m2b, j1b, j2b, ev)

            m1a, m2a, j1a, j2a, m1b, m2b, j1b, j2b, _ = lax.fori_loop(
                0, _NE // _UNROLL, estep,
                (neg, neg, zz, zz, neg, neg, zz, zz, zz),
            )
            exa = jnp.exp(m2a - m1a)
            exb = jnp.exp(m2b - m1b)
            wa = 1.0 / (1.0 + exa)
            wb = 1.0 / (1.0 + exb)
            w1v[pl.ds(ga, _L)] = wa
            w2v[pl.ds(ga, _L)] = exa * wa
            i1v[pl.ds(ga, _L)] = j1a
            i2v[pl.ds(ga, _L)] = j2a
            w1v[pl.ds(gb, _L)] = wb
            w2v[pl.ds(gb, _L)] = exb * wb
            i1v[pl.ds(gb, _L)] = j1b
            i2v[pl.ds(gb, _L)] = j2b
            return 0

        lax.fori_loop(0, _TPW // (2 * _L), group, 0)
        pltpu.sync_copy(w1v, w1_hbm.at[pl.ds(base, _TPW)])
        pltpu.sync_copy(w2v, w2_hbm.at[pl.ds(base, _TPW)])
        pltpu.sync_copy(i1v, i1_hbm.at[pl.ds(base, _TPW)])
        pltpu.sync_copy(i2v, i2_hbm.at[pl.ds(base, _TPW)])

    return sc_router


def kernel(hidden_states, gate_w, gate_b):
    T, _ = hidden_states.shape
    logits = _tc_logits(hidden_states, gate_w, gate_b)
    w1, w2, i1, i2 = _make_sc_router(T)(logits)
    weights = jnp.stack([w1, w2], axis=-1)
    idx = jnp.stack([i1, i2], axis=-1)
    return (weights, idx)
